# baseline (speedup 1.0000x reference)
import jax
import jax.numpy as jnp
from jax import lax
from jax.experimental import pallas as pl
from jax.experimental.pallas import tpu as pltpu
from jax.experimental.pallas import tpu_sc as plsc

K = 32
LANES = 16
NC = 2
NS = 16
NW = NC * NS
CHUNK = 400
ISLICE = 80
NSLICE = CHUNK // ISLICE
NGROUP = CHUNK // LANES


def _mf_body(users_hbm, items_hbm, ue_hbm, ie_hbm, ib_hbm, out_hbm,
             ustage, istage, uc0, ic0, uc1, ic1,
             urows0, irows0, bias0, out0, urows1, irows1, bias1, out1,
             sem0, sem1):
    b_per_w = users_hbm.shape[0] // NW
    L = users_hbm.shape[1]
    rows_per_chunk = CHUNK // L
    t_per_w = b_per_w * L
    n_chunks = t_per_w // CHUNK
    n_pairs = n_chunks // 2
    wid = lax.axis_index("s") * NC + lax.axis_index("c")
    wrow = wid * b_per_w
    wbase = wid * t_per_w

    pltpu.sync_copy(users_hbm.at[pl.ds(wrow, b_per_w)], ustage)
    pltpu.sync_copy(items_hbm.at[pl.ds(wrow, b_per_w)], istage)

    lane_iota = lax.iota(jnp.int32, LANES)

    def repack(c, ucbuf, icbuf):
        base = c * rows_per_chunk
        for g in range(NGROUP):
            v = g * LANES + lane_iota
            srow = lax.shift_right_logical(v * 41, 11)
            scol = v - srow * L
            st = base + srow
            row, col = (g * LANES) // ISLICE, (g * LANES) % ISLICE
            ucbuf[row, pl.ds(col, LANES)] = plsc.load_gather(ustage, [st, scol])
            icbuf[row, pl.ds(col, LANES)] = plsc.load_gather(istage, [st, scol])

    def transfers(ucbuf, icbuf, urows, irows, bias, sem):
        cps = []
        for j in range(NSLICE):
            sl = pl.ds(j * ISLICE, ISLICE)
            cps.append(pltpu.make_async_copy(ue_hbm.at[ucbuf.at[j]],
                                             urows.at[sl], sem))
            cps.append(pltpu.make_async_copy(ie_hbm.at[icbuf.at[j]],
                                             irows.at[sl], sem))
            cps.append(pltpu.make_async_copy(ib_hbm.at[icbuf.at[j]],
                                             bias.at[sl], sem))
        return cps

    def fire(ucbuf, icbuf, urows, irows, bias, sem):
        for cp in transfers(ucbuf, icbuf, urows, irows, bias, sem):
            cp.start()

    def drain(ucbuf, icbuf, urows, irows, bias, sem):
        for cp in transfers(ucbuf, icbuf, urows, irows, bias, sem):
            cp.wait()

    def compute(c, urows, irows, bias, outv):
        def group_body(g, _):
            rows = g * LANES + lane_iota
            acc = [jnp.zeros((LANES,), jnp.float32) for _ in range(4)]
            for k in range(K):
                kvec = jnp.full((LANES,), k, jnp.int32)
                u_c = plsc.load_gather(urows, [rows, kvec])
                i_c = plsc.load_gather(irows, [rows, kvec])
                acc[k % 4] = acc[k % 4] + u_c * i_c
            total = (acc[0] + acc[1]) + (acc[2] + acc[3]) \
                + bias[pl.ds(g * LANES, LANES)]
            outv[pl.ds(g * LANES, LANES)] = total
            return 0

        lax.fori_loop(0, NGROUP, group_body, 0)
        pltpu.sync_copy(outv, out_hbm.at[pl.ds(wbase + c * CHUNK, CHUNK)])

    repack(0, uc0, ic0)
    fire(uc0, ic0, urows0, irows0, bias0, sem0)
    repack(1, uc1, ic1)

    def pair_body(p, _):
        c = p * 2
        fire(uc1, ic1, urows1, irows1, bias1, sem1)
        drain(uc0, ic0, urows0, irows0, bias0, sem0)
        compute(c, urows0, irows0, bias0, out0)

        @pl.when(p < n_pairs - 1)
        def _():
            repack(c + 2, uc0, ic0)
            fire(uc0, ic0, urows0, irows0, bias0, sem0)

        drain(uc1, ic1, urows1, irows1, bias1, sem1)
        compute(c + 1, urows1, irows1, bias1, out1)

        @pl.when(p < n_pairs - 1)
        def _():
            repack(c + 3, uc1, ic1)

        return 0

    lax.fori_loop(0, n_pairs, pair_body, 0)


def kernel(users, items, user_embedding, item_embedding, item_bias):
    B, L = users.shape
    b_per_w = B // NW

    mesh = plsc.VectorSubcoreMesh(core_axis_name="c", subcore_axis_name="s",
                                  num_cores=NC, num_subcores=NS)
    run = pl.kernel(
        _mf_body,
        out_type=jax.ShapeDtypeStruct((B * L,), jnp.float32),
        mesh=mesh,
        compiler_params=pltpu.CompilerParams(needs_layout_passes=False,
                                             use_tc_tiling_on_sc=False),
        scratch_types=[
            pltpu.VMEM((b_per_w, L), jnp.int32),
            pltpu.VMEM((b_per_w, L), jnp.int32),
            pltpu.VMEM((NSLICE, ISLICE), jnp.int32),
            pltpu.VMEM((NSLICE, ISLICE), jnp.int32),
            pltpu.VMEM((NSLICE, ISLICE), jnp.int32),
            pltpu.VMEM((NSLICE, ISLICE), jnp.int32),
            pltpu.VMEM((CHUNK, K), jnp.float32),
            pltpu.VMEM((CHUNK, K), jnp.float32),
            pltpu.VMEM((CHUNK,), jnp.float32),
            pltpu.VMEM((CHUNK,), jnp.float32),
            pltpu.VMEM((CHUNK, K), jnp.float32),
            pltpu.VMEM((CHUNK, K), jnp.float32),
            pltpu.VMEM((CHUNK,), jnp.float32),
            pltpu.VMEM((CHUNK,), jnp.float32),
            pltpu.SemaphoreType.DMA,
            pltpu.SemaphoreType.DMA,
        ],
    )
    out = run(users, items, user_embedding, item_embedding,
              item_bias.reshape(-1))
    return out.reshape(B, L)


# speedup vs baseline: 1.6709x; 1.6709x over previous
import jax
import jax.numpy as jnp
from jax import lax
from jax.experimental import pallas as pl
from jax.experimental.pallas import tpu as pltpu
from jax.experimental.pallas import tpu_sc as plsc

K = 32
LANES = 16
NC = 2
NS = 16
NW = NC * NS
CHUNK = 400
ISLICE = 80
NSLICE = CHUNK // ISLICE
NGROUP = CHUNK // LANES


def _mf_body(users_hbm, items_hbm, ue_hbm, ie_hbm, ib_hbm, out_hbm,
             ustage, istage, uc0, ic0, uc1, ic1,
             urows0, irows0, bias0, out0, urows1, irows1, bias1, out1,
             sem0, sem1):
    b_per_w = users_hbm.shape[0] // NW
    L = users_hbm.shape[1]
    rows_per_chunk = CHUNK // L
    t_per_w = b_per_w * L
    n_chunks = t_per_w // CHUNK
    n_pairs = n_chunks // 2
    wid = lax.axis_index("s") * NC + lax.axis_index("c")
    wrow = wid * b_per_w
    wbase = wid * t_per_w

    pltpu.sync_copy(users_hbm.at[pl.ds(wrow, b_per_w)], ustage)
    pltpu.sync_copy(items_hbm.at[pl.ds(wrow, b_per_w)], istage)

    lane_iota = lax.iota(jnp.int32, LANES)

    def repack(c, ucbuf, icbuf):
        base = c * rows_per_chunk
        for g in range(NGROUP):
            v = g * LANES + lane_iota
            srow = lax.shift_right_logical(v * 41, 11)
            scol = v - srow * L
            st = base + srow
            row, col = (g * LANES) // ISLICE, (g * LANES) % ISLICE
            ucbuf[row, pl.ds(col, LANES)] = plsc.load_gather(ustage, [st, scol])
            icbuf[row, pl.ds(col, LANES)] = plsc.load_gather(istage, [st, scol])

    def transfers(ucbuf, icbuf, urows, irows, bias, sem):
        cps = []
        for j in range(NSLICE):
            sl = pl.ds(j * ISLICE, ISLICE)
            cps.append(pltpu.make_async_copy(ue_hbm.at[ucbuf.at[j]],
                                             urows.at[sl], sem))
            cps.append(pltpu.make_async_copy(ie_hbm.at[icbuf.at[j]],
                                             irows.at[sl], sem))
            cps.append(pltpu.make_async_copy(ib_hbm.at[icbuf.at[j]],
                                             bias.at[sl], sem))
        return cps

    def fire(ucbuf, icbuf, urows, irows, bias, sem):
        for cp in transfers(ucbuf, icbuf, urows, irows, bias, sem):
            cp.start()

    def drain(ucbuf, icbuf, urows, irows, bias, sem):
        for cp in transfers(ucbuf, icbuf, urows, irows, bias, sem):
            cp.wait()

    def compute(c, urows, irows, bias, outv):
        def group_body(g, _):
            rows = g * LANES + lane_iota
            acc = [jnp.zeros((LANES,), jnp.float32) for _ in range(4)]
            for k in range(K):
                colv = (lane_iota + k) & (K - 1)
                u_c = plsc.load_gather(urows, [rows, colv])
                i_c = plsc.load_gather(irows, [rows, colv])
                acc[k % 4] = acc[k % 4] + u_c * i_c
            total = (acc[0] + acc[1]) + (acc[2] + acc[3]) \
                + bias[pl.ds(g * LANES, LANES)]
            outv[pl.ds(g * LANES, LANES)] = total
            return 0

        lax.fori_loop(0, NGROUP, group_body, 0)
        pltpu.sync_copy(outv, out_hbm.at[pl.ds(wbase + c * CHUNK, CHUNK)])

    repack(0, uc0, ic0)
    fire(uc0, ic0, urows0, irows0, bias0, sem0)
    repack(1, uc1, ic1)

    def pair_body(p, _):
        c = p * 2
        fire(uc1, ic1, urows1, irows1, bias1, sem1)
        drain(uc0, ic0, urows0, irows0, bias0, sem0)
        compute(c, urows0, irows0, bias0, out0)

        @pl.when(p < n_pairs - 1)
        def _():
            repack(c + 2, uc0, ic0)
            fire(uc0, ic0, urows0, irows0, bias0, sem0)

        drain(uc1, ic1, urows1, irows1, bias1, sem1)
        compute(c + 1, urows1, irows1, bias1, out1)

        @pl.when(p < n_pairs - 1)
        def _():
            repack(c + 3, uc1, ic1)

        return 0

    lax.fori_loop(0, n_pairs, pair_body, 0)


def kernel(users, items, user_embedding, item_embedding, item_bias):
    B, L = users.shape
    b_per_w = B // NW

    mesh = plsc.VectorSubcoreMesh(core_axis_name="c", subcore_axis_name="s",
                                  num_cores=NC, num_subcores=NS)
    run = pl.kernel(
        _mf_body,
        out_type=jax.ShapeDtypeStruct((B * L,), jnp.float32),
        mesh=mesh,
        compiler_params=pltpu.CompilerParams(needs_layout_passes=False,
                                             use_tc_tiling_on_sc=False),
        scratch_types=[
            pltpu.VMEM((b_per_w, L), jnp.int32),
            pltpu.VMEM((b_per_w, L), jnp.int32),
            pltpu.VMEM((NSLICE, ISLICE), jnp.int32),
            pltpu.VMEM((NSLICE, ISLICE), jnp.int32),
            pltpu.VMEM((NSLICE, ISLICE), jnp.int32),
            pltpu.VMEM((NSLICE, ISLICE), jnp.int32),
            pltpu.VMEM((CHUNK, K), jnp.float32),
            pltpu.VMEM((CHUNK, K), jnp.float32),
            pltpu.VMEM((CHUNK,), jnp.float32),
            pltpu.VMEM((CHUNK,), jnp.float32),
            pltpu.VMEM((CHUNK, K), jnp.float32),
            pltpu.VMEM((CHUNK, K), jnp.float32),
            pltpu.VMEM((CHUNK,), jnp.float32),
            pltpu.VMEM((CHUNK,), jnp.float32),
            pltpu.SemaphoreType.DMA,
            pltpu.SemaphoreType.DMA,
        ],
    )
    out = run(users, items, user_embedding, item_embedding,
              item_bias.reshape(-1))
    return out.reshape(B, L)
